# baseline (speedup 1.0000x reference)
import jax
import jax.numpy as jnp
from jax.experimental import pallas as pl
from jax.experimental.pallas import tpu as pltpu

_LEAKY_ALPHA = 0.2


def _gat_fused_kernel(xt_ref, adjt_ref, wht_w_ref, f_w_ref, o_ref):
    D, N = xt_ref.shape[1], xt_ref.shape[2]
    H = f_w_ref.shape[0] // 2
    DA = D + 8
    S = 2 * D

    adjt = adjt_ref[0]
    madj = adjt > 0.0

    ones = jnp.ones((8, N), jnp.bfloat16)
    xat = jnp.concatenate([xt_ref[0].astype(jnp.bfloat16), ones], axis=0)

    wht = jnp.dot(wht_w_ref[...], xat,
                  preferred_element_type=jnp.float32).astype(jnp.bfloat16)
    f12 = jnp.dot(f_w_ref[...], xat,
                  preferred_element_type=jnp.float32)
    f1t = f12[:H]
    f2t = f12[H:]

    f2max = jnp.max(f2t, axis=1, keepdims=True)
    m = jnp.maximum(f1t + f2max, 0.0)
    u1 = f1t - m
    v1 = _LEAKY_ALPHA * f1t - m

    f2c = f2t.T
    f2c_s = _LEAKY_ALPHA * f2c

    zero = jnp.zeros((), jnp.bfloat16)
    accT = jnp.zeros((D, N), jnp.float32)
    for h in range(H):
        u = f2c[:, h:h + 1] + u1[h:h + 1, :]
        v = f2c_s[:, h:h + 1] + v1[h:h + 1, :]
        pb = jnp.where(madj, jnp.exp(jnp.maximum(u, v)).astype(jnp.bfloat16),
                       zero)
        g = jnp.dot(wht[h * S:h * S + DA], pb,
                    preferred_element_type=jnp.float32)
        r = pl.reciprocal(g[D:D + 1, :], approx=True)
        accT = accT + g[:D, :] * r

    avgT = accT * (1.0 / H)
    outT = jnp.where(avgT > 0, avgT, jnp.exp(jnp.minimum(avgT, 0.0)) - 1.0)
    o_ref[0] = outT.T.astype(o_ref.dtype)


@jax.jit
def _gat_layer(x, adj, w, a1, a2):
    B, N, D = x.shape
    H = w.shape[0]
    DA = D + 8
    S = 2 * D

    xt = jnp.swapaxes(x, 1, 2)
    adjt = jnp.swapaxes(adj, 1, 2)

    wht_w = jnp.zeros((H * S, DA), jnp.float32)
    eye8 = jnp.eye(8, dtype=jnp.float32)
    for h in range(H):
        wht_w = wht_w.at[h * S:h * S + D, :D].set(w[h].T)
        wht_w = wht_w.at[h * S + D:h * S + D + 8, D:].set(eye8)
    wht_w = wht_w.astype(jnp.bfloat16)

    wa1 = jnp.einsum('hdc,hc->hd', w, a1.reshape(H, D))
    wa2 = jnp.einsum('hdc,hc->hd', w, a2.reshape(H, D))
    f_w = jnp.concatenate([wa1, wa2], axis=0)
    f_w = jnp.concatenate([f_w, jnp.zeros((2 * H, 8), jnp.float32)],
                          axis=1).astype(jnp.bfloat16)

    return pl.pallas_call(
        _gat_fused_kernel,
        out_shape=jax.ShapeDtypeStruct((B, N, D), x.dtype),
        grid=(B,),
        in_specs=[
            pl.BlockSpec((1, D, N), lambda b: (b, 0, 0)),
            pl.BlockSpec((1, N, N), lambda b: (b, 0, 0)),
            pl.BlockSpec((H * S, DA), lambda b: (0, 0)),
            pl.BlockSpec((2 * H, DA), lambda b: (0, 0)),
        ],
        out_specs=pl.BlockSpec((1, N, D), lambda b: (b, 0, 0)),
        compiler_params=pltpu.CompilerParams(
            dimension_semantics=("parallel",)),
    )(xt, adjt, wht_w, f_w)


def kernel(x, adj, w, a1, a2):
    return _gat_layer(x, adj, w, a1, a2)


# speedup vs baseline: 1.3139x; 1.3139x over previous
import jax
import jax.numpy as jnp
from jax.experimental import pallas as pl
from jax.experimental.pallas import tpu as pltpu

_LEAKY_ALPHA = 0.2


def _gat_fused_kernel(x_ref, adj_ref, wht_w_ref, f_w_ref, f2r_w_ref, o_ref):
    N, D = x_ref.shape[1], x_ref.shape[2]
    H = f2r_w_ref.shape[1]
    DA = D + 8
    S = 2 * D

    adj = adj_ref[0]
    madj = adj > 0.0

    ones = jnp.ones((N, 8), jnp.bfloat16)
    xa = jnp.concatenate([x_ref[0].astype(jnp.bfloat16), ones], axis=1)

    wht = jax.lax.dot_general(
        wht_w_ref[...], xa, (((1,), (1,)), ((), ())),
        preferred_element_type=jnp.float32).astype(jnp.bfloat16)

    f12 = jnp.dot(xa, f_w_ref[...],
                  preferred_element_type=jnp.float32)
    f1 = f12[:, :H]
    f2 = f12[:, H:]
    f2r = jax.lax.dot_general(
        f2r_w_ref[...], xa, (((0,), (1,)), ((), ())),
        preferred_element_type=jnp.float32)
    f2r_s = _LEAKY_ALPHA * f2r

    f2max = jnp.max(f2, axis=0, keepdims=True)
    m = jnp.maximum(f1 + f2max, 0.0)
    u1 = f1 - m
    v1 = _LEAKY_ALPHA * f1 - m

    dn_aggr = (((1,), (1,)), ((), ()))
    zero = jnp.zeros((), jnp.bfloat16)
    accT = jnp.zeros((D, N), jnp.float32)
    for h in range(H):
        u = u1[:, h:h + 1] + f2r[h:h + 1, :]
        v = v1[:, h:h + 1] + f2r_s[h:h + 1, :]
        pb = jnp.where(madj, jnp.exp(jnp.maximum(u, v)).astype(jnp.bfloat16),
                       zero)
        g = jax.lax.dot_general(wht[h * S:h * S + DA], pb, dn_aggr,
                                preferred_element_type=jnp.float32)
        r = pl.reciprocal(g[D:D + 1, :], approx=True)
        accT = accT + g[:D, :] * r

    avgT = accT * (1.0 / H)
    outT = jnp.where(avgT > 0, avgT, jnp.exp(jnp.minimum(avgT, 0.0)) - 1.0)
    o_ref[0] = outT.T.astype(o_ref.dtype)


@jax.jit
def _gat_layer(x, adj, w, a1, a2):
    B, N, D = x.shape
    H = w.shape[0]
    DA = D + 8
    S = 2 * D

    wht_w = jnp.zeros((H * S, DA), jnp.float32)
    eye8 = jnp.eye(8, dtype=jnp.float32)
    for h in range(H):
        wht_w = wht_w.at[h * S:h * S + D, :D].set(w[h].T)
        wht_w = wht_w.at[h * S + D:h * S + D + 8, D:].set(eye8)
    wht_w = wht_w.astype(jnp.bfloat16)

    wa1 = jnp.einsum('hdc,hc->dh', w, a1.reshape(H, D))
    wa2 = jnp.einsum('hdc,hc->dh', w, a2.reshape(H, D))
    f_w = jnp.concatenate([wa1, wa2], axis=1)
    f_w = jnp.concatenate([f_w, jnp.zeros((8, 2 * H), jnp.float32)],
                          axis=0).astype(jnp.bfloat16)
    f2r_w = jnp.concatenate([wa2, jnp.zeros((8, H), jnp.float32)],
                            axis=0).astype(jnp.bfloat16)

    return pl.pallas_call(
        _gat_fused_kernel,
        out_shape=jax.ShapeDtypeStruct((B, N, D), x.dtype),
        grid=(B,),
        in_specs=[
            pl.BlockSpec((1, N, D), lambda b: (b, 0, 0)),
            pl.BlockSpec((1, N, N), lambda b: (b, 0, 0)),
            pl.BlockSpec((H * S, DA), lambda b: (0, 0)),
            pl.BlockSpec((DA, 2 * H), lambda b: (0, 0)),
            pl.BlockSpec((DA, H), lambda b: (0, 0)),
        ],
        out_specs=pl.BlockSpec((1, N, D), lambda b: (b, 0, 0)),
        compiler_params=pltpu.CompilerParams(
            dimension_semantics=("parallel",)),
    )(x, adj, wht_w, f_w, f2r_w)


def kernel(x, adj, w, a1, a2):
    return _gat_layer(x, adj, w, a1, a2)


# speedup vs baseline: 1.7023x; 1.2955x over previous
import jax
import jax.numpy as jnp
from jax.experimental import pallas as pl
from jax.experimental.pallas import tpu as pltpu

_LEAKY_ALPHA = 0.2


def _gat_fused_kernel(x_ref, adj_ref, wht_w_ref, f_w_ref, f2r_w_ref, o_ref):
    for bb in range(x_ref.shape[0]):
        _gat_one(x_ref, adj_ref, wht_w_ref, f_w_ref, f2r_w_ref, o_ref, bb)


def _gat_one(x_ref, adj_ref, wht_w_ref, f_w_ref, f2r_w_ref, o_ref, bb):
    N, D = x_ref.shape[1], x_ref.shape[2]
    H = f2r_w_ref.shape[1]
    DA = D + 8
    S = DA

    adj = adj_ref[bb]
    madj = adj > 0.0

    ones = jnp.ones((N, 8), jnp.bfloat16)
    xa = jnp.concatenate([x_ref[bb].astype(jnp.bfloat16), ones], axis=1)

    wht = jax.lax.dot_general(
        wht_w_ref[...], xa, (((1,), (1,)), ((), ())),
        preferred_element_type=jnp.float32).astype(jnp.bfloat16)

    f12 = jnp.dot(xa, f_w_ref[...],
                  preferred_element_type=jnp.float32)
    f1 = f12[:, :H]
    f2 = f12[:, H:]
    f2r = jax.lax.dot_general(
        f2r_w_ref[...], xa, (((0,), (1,)), ((), ())),
        preferred_element_type=jnp.float32)
    f2r_s = _LEAKY_ALPHA * f2r

    f2max = jnp.max(f2, axis=0, keepdims=True)
    m = jnp.maximum(f1 + f2max, 0.0)
    u1 = f1 - m
    v1 = _LEAKY_ALPHA * f1 - m

    dn_aggr = (((1,), (1,)), ((), ()))
    zero = jnp.zeros((), jnp.bfloat16)
    accT = jnp.zeros((D, N), jnp.float32)
    for h in range(H):
        u = u1[:, h:h + 1] + f2r[h:h + 1, :]
        v = v1[:, h:h + 1] + f2r_s[h:h + 1, :]
        pb = jnp.where(madj, jnp.exp2(jnp.maximum(u, v)).astype(jnp.bfloat16),
                       zero)
        g = jax.lax.dot_general(wht[h * S:h * S + DA], pb, dn_aggr,
                                preferred_element_type=jnp.float32)
        r = pl.reciprocal(g[D:D + 1, :], approx=True)
        accT = accT + g[:D, :] * r

    avgT = accT
    outT = jnp.where(avgT > 0, avgT, jnp.exp(jnp.minimum(avgT, 0.0)) - 1.0)
    o_ref[bb] = outT.T.astype(o_ref.dtype)


@jax.jit
def _gat_layer(x, adj, w, a1, a2):
    B, N, D = x.shape
    H = w.shape[0]
    DA = D + 8
    S = DA

    wt = jnp.concatenate([jnp.swapaxes(w, 1, 2),
                          jnp.zeros((H, D, 8), jnp.float32)], axis=2)
    bot = jnp.concatenate([jnp.zeros((8, D), jnp.float32),
                           float(H) * jnp.eye(8, dtype=jnp.float32)], axis=1)
    blocks = jnp.concatenate(
        [wt, jnp.broadcast_to(bot[None], (H, 8, DA))], axis=1)
    wht_w = blocks.reshape(H * S, DA).astype(jnp.bfloat16)

    log2e = jnp.float32(1.4426950408889634)
    wa1 = log2e * jnp.einsum('hdc,hc->dh', w, a1.reshape(H, D))
    wa2 = log2e * jnp.einsum('hdc,hc->dh', w, a2.reshape(H, D))
    f_w = jnp.concatenate([wa1, wa2], axis=1)
    f_w = jnp.concatenate([f_w, jnp.zeros((8, 2 * H), jnp.float32)],
                          axis=0).astype(jnp.bfloat16)
    f2r_w = jnp.concatenate([wa2, jnp.zeros((8, H), jnp.float32)],
                            axis=0).astype(jnp.bfloat16)

    return pl.pallas_call(
        _gat_fused_kernel,
        out_shape=jax.ShapeDtypeStruct((B, N, D), x.dtype),
        grid=(B // 4,),
        in_specs=[
            pl.BlockSpec((4, N, D), lambda b: (b, 0, 0)),
            pl.BlockSpec((4, N, N), lambda b: (b, 0, 0)),
            pl.BlockSpec((H * S, DA), lambda b: (0, 0)),
            pl.BlockSpec((DA, 2 * H), lambda b: (0, 0)),
            pl.BlockSpec((DA, H), lambda b: (0, 0)),
        ],
        out_specs=pl.BlockSpec((4, N, D), lambda b: (b, 0, 0)),
        compiler_params=pltpu.CompilerParams(
            dimension_semantics=("parallel",)),
    )(x, adj, wht_w, f_w, f2r_w)


def kernel(x, adj, w, a1, a2):
    return _gat_layer(x, adj, w, a1, a2)
